# baseline (speedup 1.0000x reference)
import functools

import jax
import jax.numpy as jnp
from jax import lax
from jax.experimental import pallas as pl
from jax.experimental.pallas import tpu as pltpu
from jax.experimental.pallas import tpu_sc as plsc

B = 16384
D = 64
H = 128
NR = 6
NRP = 8
NC = 2
NS = 16
NW = NC * NS
BPW = B // NW
V = 1000000
CW = 128
TAIL0 = (V // CW) * CW
NSLOT = 10


def _sc_gather_t(user_idx, item_idx, utab_t, itab_t):
    mesh = plsc.VectorSubcoreMesh(core_axis_name="c", subcore_axis_name="s")

    @functools.partial(
        pl.kernel,
        mesh=mesh,
        out_type=(
            jax.ShapeDtypeStruct((B * D,), jnp.float32),
            jax.ShapeDtypeStruct((B * D,), jnp.float32),
        ),
        scratch_types=[
            pltpu.VMEM((BPW + 16,), jnp.int32),
            pltpu.VMEM((NSLOT, D, CW), jnp.float32),
            pltpu.VMEM((BPW * D,), jnp.float32),
        ] + [pltpu.SemaphoreType.DMA] * (NSLOT + 1),
    )
    def gather_kernel(uidx_hbm, iidx_hbm, utab_hbm, itab_hbm,
                      uout_hbm, iout_hbm,
                      idx_v, blocks, rows, *dmasems):
        wid = lax.axis_index("s") * NC + lax.axis_index("c")
        base = wid * BPW
        iota = lax.iota(jnp.int32, 16)
        lsel = [iota == l for l in range(16)]
        sems = list(dmasems[:NSLOT])
        semw = dmasems[NSLOT]

        def col_of(vec, t):
            j = vec[t]
            q = jnp.minimum((j // CW) * CW, TAIL0 - CW)
            return pl.multiple_of(q, CW), j - (j // CW) * CW

        def phase(idx_hbm, tab_hbm, out_hbm):
            pltpu.sync_copy(idx_hbm.at[wid], idx_v.at[pl.ds(0, BPW)])

            def fetch(vec, t, slot):
                q, _ = col_of(vec, t)
                pltpu.async_copy(
                    tab_hbm.at[:, pl.ds(q, CW)],
                    blocks.at[slot], sems[slot])

            def extract(vec, t, slot, kpos):
                _, r = col_of(vec, t)
                pltpu.make_async_copy(
                    tab_hbm.at[:, pl.ds(0, CW)],
                    blocks.at[slot], sems[slot]).wait()
                rq = pl.multiple_of((r // 16) * 16, 16)
                rsp = iota * 0 + (r - rq)
                for q in range(4):
                    acc = jnp.zeros((16,), jnp.float32)
                    for l in range(16):
                        span = blocks[slot, q * 16 + l, pl.ds(rq, 16)]
                        v = span[rsp]
                        acc = jnp.where(lsel[l], v, acc)
                    rows[pl.ds(kpos * D + q * 16, 16)] = acc

            vec0 = idx_v[pl.ds(0, 16)]
            for t in range(NSLOT):
                fetch(vec0, t, t)

            @pl.loop(0, BPW, step=16)
            def body(g):
                vec = idx_v[pl.ds(g, 16)]
                nvec = idx_v[pl.ds(g + 16, 16)]
                for t in range(16):
                    slot = t % NSLOT
                    extract(vec, t, slot, g + t)

                    @pl.when(g + t + NSLOT < BPW)
                    def _():
                        if t + NSLOT < 16:
                            fetch(vec, t + NSLOT, slot)
                        else:
                            fetch(nvec, t + NSLOT - 16, slot)

            cp = pltpu.async_copy(
                rows, out_hbm.at[pl.ds(base * D, BPW * D)], semw)
            cp.wait()

        phase(uidx_hbm, utab_hbm, uout_hbm)
        phase(iidx_hbm, itab_hbm, iout_hbm)

    return gather_kernel(
        user_idx.reshape(NW, BPW),
        item_idx.reshape(NW, BPW),
        utab_t, itab_t)


BLK = 2048
NB = B // BLK


def _tc_body(u_ref, i_ref, u3_ref, i3_ref, r_ref, ut_ref, it_ref,
             rt_ref, w_ref, b_ref, h_ref, ue_ref, ie_ref, re_ref):
    def tail_embed(idx, tail_tab):
        t2 = (idx - TAIL0).reshape(BLK, 1)
        oh = ((t2 == lax.broadcasted_iota(jnp.int32, (BLK, D), 1))
              .astype(jnp.float32) * (t2 >= 0).astype(jnp.float32))
        return jnp.dot(oh, tail_tab, preferred_element_type=jnp.float32)

    def not_tail(idx):
        return (idx.reshape(BLK, 1) < TAIL0).astype(jnp.float32)

    ue = (u_ref[...] * not_tail(u3_ref[0, 0, :])
          + tail_embed(u3_ref[0, 0, :], ut_ref[...]))
    ie = (i_ref[...] * not_tail(i3_ref[0, 0, :])
          + tail_embed(i3_ref[0, 0, :], it_ref[...]))
    r = r_ref[0, 0, :]
    onehot = (r.reshape(BLK, 1)
              == lax.broadcasted_iota(jnp.int32, (BLK, NRP), 1)
              ).astype(jnp.float32)
    re = jnp.dot(onehot, rt_ref[...], preferred_element_type=jnp.float32)
    cat = jnp.concatenate([ue, ie, re], axis=-1)
    h_ref[...] = jnp.tanh(
        jnp.dot(cat, w_ref[...], preferred_element_type=jnp.float32)
        + b_ref[...])
    ue_ref[...] = ue
    ie_ref[...] = ie
    re_ref[...] = re


def _tc_encode(u_sc, i_sc, user, item, rating,
               utail, itail, rating_table, W, b):
    rt_pad = jnp.pad(rating_table, ((0, NRP - NR), (0, 0)))
    u3 = user.reshape(NB, 1, BLK)
    i3 = item.reshape(NB, 1, BLK)
    r3 = rating.reshape(NB, 1, BLK)
    return pl.pallas_call(
        _tc_body,
        grid=(NB,),
        in_specs=[
            pl.BlockSpec((BLK, D), lambda i: (i, 0)),
            pl.BlockSpec((BLK, D), lambda i: (i, 0)),
            pl.BlockSpec((1, 1, BLK), lambda i: (i, 0, 0)),
            pl.BlockSpec((1, 1, BLK), lambda i: (i, 0, 0)),
            pl.BlockSpec((1, 1, BLK), lambda i: (i, 0, 0)),
            pl.BlockSpec((D, D), lambda i: (0, 0)),
            pl.BlockSpec((D, D), lambda i: (0, 0)),
            pl.BlockSpec((NRP, D), lambda i: (0, 0)),
            pl.BlockSpec((3 * D, H), lambda i: (0, 0)),
            pl.BlockSpec((1, H), lambda i: (0, 0)),
        ],
        out_specs=[
            pl.BlockSpec((BLK, H), lambda i: (i, 0)),
            pl.BlockSpec((BLK, D), lambda i: (i, 0)),
            pl.BlockSpec((BLK, D), lambda i: (i, 0)),
            pl.BlockSpec((BLK, D), lambda i: (i, 0)),
        ],
        out_shape=[
            jax.ShapeDtypeStruct((B, H), jnp.float32),
            jax.ShapeDtypeStruct((B, D), jnp.float32),
            jax.ShapeDtypeStruct((B, D), jnp.float32),
            jax.ShapeDtypeStruct((B, D), jnp.float32),
        ],
    )(u_sc, i_sc, u3, i3, r3, utail, itail, rt_pad, W, b.reshape(1, H))


def kernel(user, item, rating, user_table, item_table, rating_table, W, b):
    u_flat, i_flat = _sc_gather_t(user, item, user_table.T, item_table.T)
    hidden, ue, ie, re = _tc_encode(
        u_flat.reshape(B, D), i_flat.reshape(B, D), user, item, rating,
        user_table[TAIL0:], item_table[TAIL0:], rating_table, W, b)
    return (hidden, ue, ie, re)


# speedup vs baseline: 1.0175x; 1.0175x over previous
import functools

import jax
import jax.numpy as jnp
from jax import lax
from jax.experimental import pallas as pl
from jax.experimental.pallas import tpu as pltpu
from jax.experimental.pallas import tpu_sc as plsc

B = 16384
D = 64
H = 128
NR = 6
NRP = 8
NC = 2
NS = 16
NW = NC * NS
BPW = B // NW
V = 1000000
CW = 128
TAIL0 = (V // CW) * CW
NSLOT = 8


def _sc_gather_t(user_idx, item_idx, utab_t, itab_t):
    mesh = plsc.VectorSubcoreMesh(core_axis_name="c", subcore_axis_name="s")

    @functools.partial(
        pl.kernel,
        mesh=mesh,
        out_type=(
            jax.ShapeDtypeStruct((B * D,), jnp.float32),
            jax.ShapeDtypeStruct((B * D,), jnp.float32),
        ),
        scratch_types=[
            pltpu.VMEM((BPW + 16,), jnp.int32),
            pltpu.VMEM((NSLOT, D, CW), jnp.float32),
            pltpu.VMEM((BPW * D,), jnp.float32),
        ] + [pltpu.SemaphoreType.DMA] * (NSLOT + 1),
    )
    def gather_kernel(uidx_hbm, iidx_hbm, utab_hbm, itab_hbm,
                      uout_hbm, iout_hbm,
                      idx_v, blocks, rows, *dmasems):
        wid = lax.axis_index("s") * NC + lax.axis_index("c")
        base = wid * BPW
        iota = lax.iota(jnp.int32, 16)
        lsel = [iota == l for l in range(16)]
        sems = list(dmasems[:NSLOT])
        semw = dmasems[NSLOT]

        def col_of(vec, t):
            j = vec[t]
            q = jnp.minimum((j // CW) * CW, TAIL0 - CW)
            return pl.multiple_of(q, CW), j - (j // CW) * CW

        def phase(idx_hbm, tab_hbm, out_hbm):
            pltpu.sync_copy(idx_hbm.at[wid], idx_v.at[pl.ds(0, BPW)])

            def fetch(vec, t, slot):
                q, _ = col_of(vec, t)
                pltpu.async_copy(
                    tab_hbm.at[:, pl.ds(q, CW)],
                    blocks.at[slot], sems[slot])

            def extract(vec, t, slot, kpos):
                _, r = col_of(vec, t)
                pltpu.make_async_copy(
                    tab_hbm.at[:, pl.ds(0, CW)],
                    blocks.at[slot], sems[slot]).wait()
                rq = pl.multiple_of((r // 16) * 16, 16)
                rsp = iota * 0 + (r - rq)
                for q in range(4):
                    acc = jnp.zeros((16,), jnp.float32)
                    for l in range(16):
                        span = blocks[slot, q * 16 + l, pl.ds(rq, 16)]
                        v = span[rsp]
                        acc = jnp.where(lsel[l], v, acc)
                    rows[pl.ds(kpos * D + q * 16, 16)] = acc

            vec0 = idx_v[pl.ds(0, 16)]
            for t in range(NSLOT):
                fetch(vec0, t, t)

            @pl.loop(0, BPW, step=16)
            def body(g):
                vec = idx_v[pl.ds(g, 16)]
                nvec = idx_v[pl.ds(g + 16, 16)]
                for t in range(16):
                    slot = t % NSLOT
                    extract(vec, t, slot, g + t)

                    @pl.when(g + t + NSLOT < BPW)
                    def _():
                        if t + NSLOT < 16:
                            fetch(vec, t + NSLOT, slot)
                        else:
                            fetch(nvec, t + NSLOT - 16, slot)

            cp = pltpu.async_copy(
                rows, out_hbm.at[pl.ds(base * D, BPW * D)], semw)
            cp.wait()

        phase(uidx_hbm, utab_hbm, uout_hbm)
        phase(iidx_hbm, itab_hbm, iout_hbm)

    return gather_kernel(
        user_idx.reshape(NW, BPW),
        item_idx.reshape(NW, BPW),
        utab_t, itab_t)


BLK = 2048
NB = B // BLK


def _tc_body(u_ref, i_ref, u3_ref, i3_ref, r_ref, ut_ref, it_ref,
             rt_ref, w_ref, b_ref, h_ref, ue_ref, ie_ref, re_ref):
    def tail_embed(idx, tail_tab):
        t2 = (idx - TAIL0).reshape(BLK, 1)
        oh = ((t2 == lax.broadcasted_iota(jnp.int32, (BLK, D), 1))
              .astype(jnp.float32) * (t2 >= 0).astype(jnp.float32))
        return jnp.dot(oh, tail_tab, preferred_element_type=jnp.float32)

    def not_tail(idx):
        return (idx.reshape(BLK, 1) < TAIL0).astype(jnp.float32)

    ue = (u_ref[...] * not_tail(u3_ref[0, 0, :])
          + tail_embed(u3_ref[0, 0, :], ut_ref[...]))
    ie = (i_ref[...] * not_tail(i3_ref[0, 0, :])
          + tail_embed(i3_ref[0, 0, :], it_ref[...]))
    r = r_ref[0, 0, :]
    onehot = (r.reshape(BLK, 1)
              == lax.broadcasted_iota(jnp.int32, (BLK, NRP), 1)
              ).astype(jnp.float32)
    re = jnp.dot(onehot, rt_ref[...], preferred_element_type=jnp.float32)
    cat = jnp.concatenate([ue, ie, re], axis=-1)
    h_ref[...] = jnp.tanh(
        jnp.dot(cat, w_ref[...], preferred_element_type=jnp.float32)
        + b_ref[...])
    ue_ref[...] = ue
    ie_ref[...] = ie
    re_ref[...] = re


def _tc_encode(u_sc, i_sc, user, item, rating,
               utail, itail, rating_table, W, b):
    rt_pad = jnp.pad(rating_table, ((0, NRP - NR), (0, 0)))
    u3 = user.reshape(NB, 1, BLK)
    i3 = item.reshape(NB, 1, BLK)
    r3 = rating.reshape(NB, 1, BLK)
    return pl.pallas_call(
        _tc_body,
        grid=(NB,),
        in_specs=[
            pl.BlockSpec((BLK, D), lambda i: (i, 0)),
            pl.BlockSpec((BLK, D), lambda i: (i, 0)),
            pl.BlockSpec((1, 1, BLK), lambda i: (i, 0, 0)),
            pl.BlockSpec((1, 1, BLK), lambda i: (i, 0, 0)),
            pl.BlockSpec((1, 1, BLK), lambda i: (i, 0, 0)),
            pl.BlockSpec((D, D), lambda i: (0, 0)),
            pl.BlockSpec((D, D), lambda i: (0, 0)),
            pl.BlockSpec((NRP, D), lambda i: (0, 0)),
            pl.BlockSpec((3 * D, H), lambda i: (0, 0)),
            pl.BlockSpec((1, H), lambda i: (0, 0)),
        ],
        out_specs=[
            pl.BlockSpec((BLK, H), lambda i: (i, 0)),
            pl.BlockSpec((BLK, D), lambda i: (i, 0)),
            pl.BlockSpec((BLK, D), lambda i: (i, 0)),
            pl.BlockSpec((BLK, D), lambda i: (i, 0)),
        ],
        out_shape=[
            jax.ShapeDtypeStruct((B, H), jnp.float32),
            jax.ShapeDtypeStruct((B, D), jnp.float32),
            jax.ShapeDtypeStruct((B, D), jnp.float32),
            jax.ShapeDtypeStruct((B, D), jnp.float32),
        ],
    )(u_sc, i_sc, u3, i3, r3, utail, itail, rt_pad, W, b.reshape(1, H))


def kernel(user, item, rating, user_table, item_table, rating_table, W, b):
    u_flat, i_flat = _sc_gather_t(user, item, user_table.T, item_table.T)
    hidden, ue, ie, re = _tc_encode(
        u_flat.reshape(B, D), i_flat.reshape(B, D), user, item, rating,
        user_table[TAIL0:], item_table[TAIL0:], rating_table, W, b)
    return (hidden, ue, ie, re)
